# Initial kernel scaffold; baseline (speedup 1.0000x reference)
import functools

import jax
import jax.numpy as jnp
from jax import lax
from jax.experimental import pallas as pl
from jax.experimental.pallas import tpu as pltpu
from jax.experimental.pallas import tpu_sc as plsc

N = 10000
E = 320000
D = 128
H1 = 16
H2 = 32

NC = 2
NS = 16
CH = 128
NCH = 80
EPT = CH * NCH
E_PAD = EPT * NC * NS
N_PAD = 10016
RPW = N_PAD // NS

_mesh = plsc.VectorSubcoreMesh(core_axis_name="c", subcore_axis_name="s")


def _seg_sum(y, srcp, dstp, zeros, H):
  @functools.partial(
      pl.kernel,
      out_type=jax.ShapeDtypeStruct((NC, N_PAD, H), jnp.float32),
      mesh=_mesh,
      scratch_types=[
          pltpu.VMEM((NCH, CH), jnp.int32),
          pltpu.VMEM((NCH, CH), jnp.int32),
          pltpu.VMEM((CH, H), jnp.float32),
          pltpu.VMEM((CH, H), jnp.float32),
          pltpu.VMEM_SHARED((N_PAD, H), jnp.float32),
          pltpu.SemaphoreType.DMA,
          pltpu.SemaphoreType.DMA,
          pltpu.SemaphoreType.DMA,
      ],
  )
  def k(y_hbm, src_hbm, dst_hbm, z_hbm, out_hbm,
        src_v, dst_v, rows_a, rows_b, acc_sh, sem_a, sem_b, sem_c):
    c = lax.axis_index("c")
    s = lax.axis_index("s")
    row0 = s * RPW
    h0 = pltpu.async_copy(
        z_hbm.at[pl.ds(row0, RPW)], acc_sh.at[pl.ds(row0, RPW)], sem_c)
    h1 = pltpu.async_copy(src_hbm.at[c, s], src_v, sem_a)
    h2 = pltpu.async_copy(dst_hbm.at[c, s], dst_v, sem_b)
    h0.wait()
    h1.wait()
    h2.wait()
    plsc.subcore_barrier()

    pltpu.async_copy(y_hbm.at[src_v.at[0]], rows_a, sem_a)

    @pl.loop(0, NCH, step=2)
    def _(j):
      pltpu.make_async_copy(y_hbm.at[src_v.at[0]], rows_a, sem_a).wait()
      pltpu.async_copy(y_hbm.at[src_v.at[j + 1]], rows_b, sem_b)
      pltpu.sync_copy(rows_a, acc_sh.at[dst_v.at[j]], add=True)
      pltpu.make_async_copy(y_hbm.at[src_v.at[0]], rows_b, sem_b).wait()

      @pl.when(j + 2 < NCH)
      def _():
        pltpu.async_copy(y_hbm.at[src_v.at[j + 2]], rows_a, sem_a)

      pltpu.sync_copy(rows_b, acc_sh.at[dst_v.at[j + 1]], add=True)

    plsc.subcore_barrier()
    pltpu.sync_copy(
        acc_sh.at[pl.ds(row0, RPW)], out_hbm.at[c, pl.ds(row0, RPW)])

  return k(y, srcp, dstp, zeros)


def _deg_count(dstp, ones, zeros):
  @functools.partial(
      pl.kernel,
      out_type=jax.ShapeDtypeStruct((NC, N_PAD, H1), jnp.float32),
      mesh=_mesh,
      scratch_types=[
          pltpu.VMEM((NCH, CH), jnp.int32),
          pltpu.VMEM((CH, H1), jnp.float32),
          pltpu.VMEM_SHARED((N_PAD, H1), jnp.float32),
          pltpu.SemaphoreType.DMA,
          pltpu.SemaphoreType.DMA,
      ],
  )
  def k(dst_hbm, ones_hbm, z_hbm, out_hbm, dst_v, ones_v, acc_sh, sem_a, sem_b):
    c = lax.axis_index("c")
    s = lax.axis_index("s")
    row0 = s * RPW
    h0 = pltpu.async_copy(
        z_hbm.at[pl.ds(row0, RPW)], acc_sh.at[pl.ds(row0, RPW)], sem_a)
    h1 = pltpu.async_copy(dst_hbm.at[c, s], dst_v, sem_b)
    h2 = pltpu.async_copy(ones_hbm, ones_v, sem_b)
    h0.wait()
    h1.wait()
    h2.wait()
    plsc.subcore_barrier()

    @pl.loop(0, NCH)
    def _(j):
      pltpu.sync_copy(ones_v, acc_sh.at[dst_v.at[j]], add=True)

    plsc.subcore_barrier()
    pltpu.sync_copy(
        acc_sh.at[pl.ds(row0, RPW)], out_hbm.at[c, pl.ds(row0, RPW)])

  return k(dstp, ones, zeros)


RB = 2000
G = N // RB


def _tc_xw1(x, W1):
  def body(x_ref, w_ref, o_ref):
    o_ref[...] = jnp.dot(x_ref[...], w_ref[...],
                         preferred_element_type=jnp.float32,
                         precision=lax.Precision.HIGHEST)

  return pl.pallas_call(
      body,
      grid=(G,),
      in_specs=[pl.BlockSpec((RB, D), lambda i: (i, 0)),
                pl.BlockSpec((D, H1), lambda i: (0, 0))],
      out_specs=pl.BlockSpec((RB, H1), lambda i: (i, 0)),
      out_shape=jax.ShapeDtypeStruct((N, H1), jnp.float32),
  )(x, W1)


def _tc_y1(degp, xw1):
  def body(d_ref, xw_ref, y_ref, dinv_ref):
    deg = d_ref[0] + d_ref[1] + 1.0
    dinv = lax.rsqrt(deg)
    dinv_ref[...] = dinv
    y_ref[...] = dinv * xw_ref[...]

  return pl.pallas_call(
      body,
      grid=(G,),
      in_specs=[pl.BlockSpec((2, RB, H1), lambda i: (0, i, 0)),
                pl.BlockSpec((RB, H1), lambda i: (i, 0))],
      out_specs=[pl.BlockSpec((RB, H1), lambda i: (i, 0)),
                 pl.BlockSpec((RB, H1), lambda i: (i, 0))],
      out_shape=[jax.ShapeDtypeStruct((N, H1), jnp.float32),
                 jax.ShapeDtypeStruct((N, H1), jnp.float32)],
  )(degp, xw1)


def _tc_layer2(acc1p, xw1, dinv, b1, W2):
  def body(a_ref, xw_ref, d_ref, b1_ref, w2_ref, y2_ref, xw2_ref):
    dv = d_ref[...]
    h1 = dv * (a_ref[0] + a_ref[1]) + dv * dv * xw_ref[...] + b1_ref[...]
    h1 = jnp.maximum(h1, 0.0)
    xw2 = jnp.dot(h1, w2_ref[...], preferred_element_type=jnp.float32,
                  precision=lax.Precision.HIGHEST)
    xw2_ref[...] = xw2
    d32 = jnp.concatenate([dv, dv], axis=1)
    y2_ref[...] = d32 * xw2

  return pl.pallas_call(
      body,
      grid=(G,),
      in_specs=[pl.BlockSpec((2, RB, H1), lambda i: (0, i, 0)),
                pl.BlockSpec((RB, H1), lambda i: (i, 0)),
                pl.BlockSpec((RB, H1), lambda i: (i, 0)),
                pl.BlockSpec((1, H1), lambda i: (0, 0)),
                pl.BlockSpec((H1, H2), lambda i: (0, 0))],
      out_specs=[pl.BlockSpec((RB, H2), lambda i: (i, 0)),
                 pl.BlockSpec((RB, H2), lambda i: (i, 0))],
      out_shape=[jax.ShapeDtypeStruct((N, H2), jnp.float32),
                 jax.ShapeDtypeStruct((N, H2), jnp.float32)],
  )(acc1p, xw1, dinv, b1, W2)


def _tc_out(acc2p, xw2, dinv, b2, w3t, b3):
  def body(a_ref, xw_ref, d_ref, b2_ref, w3_ref, b3_ref, o_ref):
    dv = d_ref[...]
    d32 = jnp.concatenate([dv, dv], axis=1)
    h2 = d32 * (a_ref[0] + a_ref[1]) + d32 * d32 * xw_ref[...] + b2_ref[...]
    o_ref[...] = (jnp.sum(h2 * w3_ref[...], axis=1, keepdims=True)
                  + b3_ref[0, 0])

  return pl.pallas_call(
      body,
      grid=(G,),
      in_specs=[pl.BlockSpec((2, RB, H2), lambda i: (0, i, 0)),
                pl.BlockSpec((RB, H2), lambda i: (i, 0)),
                pl.BlockSpec((RB, H1), lambda i: (i, 0)),
                pl.BlockSpec((1, H2), lambda i: (0, 0)),
                pl.BlockSpec((1, H2), lambda i: (0, 0)),
                pl.BlockSpec((1, 1), lambda i: (0, 0))],
      out_specs=pl.BlockSpec((RB, 1), lambda i: (i, 0)),
      out_shape=jax.ShapeDtypeStruct((N, 1), jnp.float32),
  )(acc2p, xw2, dinv, b2, w3t, b3)


@jax.jit
def _run(x, edge_index, W1, b1, W2, b2, W3, b3):
  src = edge_index[0]
  dst = edge_index[1]
  pad = E_PAD - E
  srcp = jnp.concatenate(
      [src, jnp.zeros((pad,), src.dtype)]).reshape(NC, NS, NCH, CH)
  dstp = jnp.concatenate(
      [dst, jnp.full((pad,), N, dst.dtype)]).reshape(NC, NS, NCH, CH)
  z16 = jnp.zeros((N_PAD, H1), jnp.float32)
  z32 = jnp.zeros((N_PAD, H2), jnp.float32)
  ones = jnp.ones((CH, H1), jnp.float32)

  degp = _deg_count(dstp, ones, z16)
  xw1 = _tc_xw1(x, W1)
  y1, dinv = _tc_y1(degp, xw1)
  acc1p = _seg_sum(y1, srcp, dstp, z16, H1)
  y2, xw2 = _tc_layer2(acc1p, xw1, dinv, b1.reshape(1, H1), W2)
  acc2p = _seg_sum(y2, srcp, dstp, z32, H2)
  return _tc_out(acc2p, xw2, dinv, b2.reshape(1, H2),
                 W3.reshape(1, H2), b3.reshape(1, 1))


def kernel(x, edge_index, W1, b1, W2, b2, W3, b3):
  return _run(x, edge_index, W1, b1, W2, b2, W3, b3)


# speedup vs baseline: 51.0824x; 51.0824x over previous
import functools

import jax
import jax.numpy as jnp
from jax import lax
from jax.experimental import pallas as pl
from jax.experimental.pallas import tpu as pltpu
from jax.experimental.pallas import tpu_sc as plsc

N = 10000
E = 320000
D = 128
H1 = 16
H2 = 32

NC = 2
NS = 16
CH = 128
NCH = 80
EPT = CH * NCH
E_PAD = EPT * NC * NS
N_PAD = 10240
NP8 = N_PAD // 8
TPW = NP8 // NS
RPW = N_PAD // NS

_mesh = plsc.VectorSubcoreMesh(core_axis_name="c", subcore_axis_name="s")
_sc_params = pltpu.CompilerParams(use_tc_tiling_on_sc=False)


def _unpack_rows(v128, v16):
  @pl.loop(0, TPW)
  def _(g):
    for k in range(8):
      v16[g * 8 + k, :] = v128[g, pl.ds(k * 16, 16)]


def _pack_rows(v16, v128):
  @pl.loop(0, TPW)
  def _(g):
    for k in range(8):
      v128[g, pl.ds(k * 16, 16)] = v16[g * 8 + k, :]


def _zero_v16(v16):
  z = jnp.zeros((16,), jnp.float32)
  @pl.loop(0, RPW)
  def _(i):
    v16[i, :] = z


def _seg_sum(tables, srcp, dstp):
  nt = len(tables)

  @functools.partial(
      pl.kernel,
      out_type=[jax.ShapeDtypeStruct((NC, NP8, 128), jnp.float32)] * nt,
      mesh=_mesh,
      compiler_params=_sc_params,
      scratch_types=(
          [pltpu.VMEM((NCH, CH), jnp.int32)] * 2
          + [pltpu.VMEM((TPW, 128), jnp.float32)]
          + [pltpu.VMEM((RPW, 16), jnp.float32)]
          + [pltpu.VMEM((CH, 16), jnp.float32)] * (2 * nt)
          + [pltpu.VMEM_SHARED((N_PAD, 16), jnp.float32)] * (2 * nt)
          + [pltpu.SemaphoreType.DMA] * (2 * nt + 1)
      ),
  )
  def k(*refs):
    y_hbm = refs[:nt]
    src_hbm, dst_hbm = refs[nt], refs[nt + 1]
    out_hbm = refs[nt + 2:2 * nt + 2]
    r = 2 * nt + 2
    src_v, dst_v = refs[r], refs[r + 1]
    v128, v16 = refs[r + 2], refs[r + 3]
    rows_a = refs[r + 4:r + 4 + nt]
    rows_b = refs[r + 4 + nt:r + 4 + 2 * nt]
    r += 4 + 2 * nt
    table_sh = refs[r:r + nt]
    acc_sh = refs[r + nt:r + 2 * nt]
    r += 2 * nt
    sem_a = refs[r:r + nt]
    sem_b = refs[r + nt:r + 2 * nt]
    sem_c = refs[r + 2 * nt]

    c = lax.axis_index("c")
    s = lax.axis_index("s")
    prow0 = s * TPW
    nrow0 = s * RPW

    h1 = pltpu.async_copy(src_hbm.at[c, s], src_v, sem_a[0])
    h2 = pltpu.async_copy(dst_hbm.at[c, s], dst_v, sem_b[0])

    for t in range(nt):
      pltpu.async_copy(
          y_hbm[t].at[pl.ds(prow0, TPW)], v128, sem_c).wait()
      _unpack_rows(v128, v16)
      pltpu.async_copy(
          v16, table_sh[t].at[pl.ds(nrow0, RPW)], sem_c).wait()

    _zero_v16(v16)
    for t in range(nt):
      pltpu.async_copy(
          v16, acc_sh[t].at[pl.ds(nrow0, RPW)], sem_c).wait()

    h1.wait()
    h2.wait()
    plsc.subcore_barrier()

    @pl.loop(0, NCH, step=2)
    def _(j):
      ha = [pltpu.async_copy(table_sh[t].at[src_v.at[j]], rows_a[t],
                             sem_a[t]) for t in range(nt)]
      hb = [pltpu.async_copy(table_sh[t].at[src_v.at[j + 1]], rows_b[t],
                             sem_b[t]) for t in range(nt)]
      for t in range(nt):
        ha[t].wait()
        pltpu.sync_copy(rows_a[t], acc_sh[t].at[dst_v.at[j]], add=True)
      for t in range(nt):
        hb[t].wait()
        pltpu.sync_copy(rows_b[t], acc_sh[t].at[dst_v.at[j + 1]], add=True)

    plsc.subcore_barrier()

    for t in range(nt):
      pltpu.async_copy(
          acc_sh[t].at[pl.ds(nrow0, RPW)], v16, sem_c).wait()
      _pack_rows(v16, v128)
      pltpu.async_copy(
          v128, out_hbm[t].at[c, pl.ds(prow0, TPW)], sem_c).wait()

  outs = k(*tables, srcp, dstp)
  return list(outs) if isinstance(outs, (list, tuple)) else [outs]


def _deg_count(dstp):
  @functools.partial(
      pl.kernel,
      out_type=jax.ShapeDtypeStruct((NC, NP8, 128), jnp.float32),
      mesh=_mesh,
      compiler_params=_sc_params,
      scratch_types=[
          pltpu.VMEM((NCH, CH), jnp.int32),
          pltpu.VMEM((TPW, 128), jnp.float32),
          pltpu.VMEM((RPW, 16), jnp.float32),
          pltpu.VMEM((CH, 16), jnp.float32),
          pltpu.VMEM_SHARED((N_PAD, 16), jnp.float32),
          pltpu.SemaphoreType.DMA,
      ],
  )
  def k(dst_hbm, out_hbm, dst_v, v128, v16, ones_v, acc_sh, sem):
    c = lax.axis_index("c")
    s = lax.axis_index("s")
    prow0 = s * TPW
    nrow0 = s * RPW

    h1 = pltpu.async_copy(dst_hbm.at[c, s], dst_v, sem)

    one = jnp.ones((16,), jnp.float32)
    @pl.loop(0, CH)
    def _(i):
      ones_v[i, :] = one

    _zero_v16(v16)
    pltpu.async_copy(v16, acc_sh.at[pl.ds(nrow0, RPW)], sem).wait()
    h1.wait()
    plsc.subcore_barrier()

    @pl.loop(0, NCH)
    def _(j):
      pltpu.sync_copy(ones_v, acc_sh.at[dst_v.at[j]], add=True)

    plsc.subcore_barrier()

    pltpu.async_copy(acc_sh.at[pl.ds(nrow0, RPW)], v16, sem).wait()
    _pack_rows(v16, v128)
    pltpu.async_copy(v128, out_hbm.at[c, pl.ds(prow0, TPW)], sem).wait()

  return k(dstp)


_HP = lax.Precision.HIGHEST


def _tc_xw1(x_r, w1bd):
  def body(x_ref, w_ref, o_ref):
    o_ref[...] = jnp.dot(x_ref[...], w_ref[...],
                         preferred_element_type=jnp.float32, precision=_HP)

  return pl.pallas_call(
      body,
      out_shape=jax.ShapeDtypeStruct((NP8, 128), jnp.float32),
  )(x_r, w1bd)


def _tc_y1(degp, xw1p):
  def body(d_ref, xw_ref, y_ref, dinv_ref):
    dinv = lax.rsqrt(d_ref[0] + d_ref[1] + 1.0)
    dinv_ref[...] = dinv
    y_ref[...] = dinv * xw_ref[...]

  return pl.pallas_call(
      body,
      out_shape=[jax.ShapeDtypeStruct((NP8, 128), jnp.float32)] * 2,
  )(degp, xw1p)


def _tc_layer2(acc1p, xw1p, dinv, b1t, w2lbd, w2rbd):
  def body(a_ref, xw_ref, d_ref, b1_ref, wl_ref, wr_ref,
           y2l_ref, y2r_ref, xw2l_ref, xw2r_ref):
    dv = d_ref[...]
    h1 = dv * (a_ref[0] + a_ref[1]) + dv * dv * xw_ref[...] + b1_ref[...]
    h1 = jnp.maximum(h1, 0.0)
    xw2l = jnp.dot(h1, wl_ref[...], preferred_element_type=jnp.float32,
                   precision=_HP)
    xw2r = jnp.dot(h1, wr_ref[...], preferred_element_type=jnp.float32,
                   precision=_HP)
    xw2l_ref[...] = xw2l
    xw2r_ref[...] = xw2r
    y2l_ref[...] = dv * xw2l
    y2r_ref[...] = dv * xw2r

  return pl.pallas_call(
      body,
      out_shape=[jax.ShapeDtypeStruct((NP8, 128), jnp.float32)] * 4,
  )(acc1p, xw1p, dinv, b1t, w2lbd, w2rbd)


def _tc_out(acc2lp, acc2rp, xw2l, xw2r, dinv, b2lt, b2rt, w3lt, w3rt, sel, b3):
  def body(al_ref, ar_ref, xl_ref, xr_ref, d_ref, b2l_ref, b2r_ref,
           w3l_ref, w3r_ref, s_ref, b3_ref, o_ref):
    dv = d_ref[...]
    dv2 = dv * dv
    h2l = dv * (al_ref[0] + al_ref[1]) + dv2 * xl_ref[...] + b2l_ref[...]
    h2r = dv * (ar_ref[0] + ar_ref[1]) + dv2 * xr_ref[...] + b2r_ref[...]
    t = h2l * w3l_ref[...] + h2r * w3r_ref[...]
    o_ref[...] = jnp.dot(t, s_ref[...], preferred_element_type=jnp.float32,
                         precision=_HP) + b3_ref[0, 0]

  return pl.pallas_call(
      body,
      out_shape=jax.ShapeDtypeStruct((NP8, 8), jnp.float32),
  )(acc2lp, acc2rp, xw2l, xw2r, dinv, b2lt, b2rt, w3lt, w3rt, sel, b3)


def _blockdiag8(W):
  din, dout = W.shape
  z = jnp.zeros((8 * din, 8 * dout), W.dtype)
  for k in range(8):
    z = lax.dynamic_update_slice(z, W, (din * k, dout * k))
  return z


@jax.jit
def _run(x, edge_index, W1, b1, W2, b2, W3, b3):
  src = edge_index[0]
  dst = edge_index[1]
  pad = E_PAD - E
  srcp = jnp.concatenate(
      [src, jnp.zeros((pad,), src.dtype)]).reshape(NC, NS, NCH, CH)
  dstp = jnp.concatenate(
      [dst, jnp.full((pad,), N, dst.dtype)]).reshape(NC, NS, NCH, CH)

  x_r = jnp.pad(x.reshape(N // 8, 8 * D), ((0, NP8 - N // 8), (0, 0)))
  w1bd = _blockdiag8(W1)
  w2lbd = _blockdiag8(W2[:, :16])
  w2rbd = _blockdiag8(W2[:, 16:])
  b1t = jnp.tile(b1, 8).reshape(1, 128)
  b2lt = jnp.tile(b2[:16], 8).reshape(1, 128)
  b2rt = jnp.tile(b2[16:], 8).reshape(1, 128)
  w3lt = jnp.tile(W3[:16, 0], 8).reshape(1, 128)
  w3rt = jnp.tile(W3[16:, 0], 8).reshape(1, 128)
  sel = jnp.repeat(jnp.eye(8, dtype=jnp.float32), 16, axis=0)
  b3r = b3.reshape(1, 1)

  degp = _deg_count(dstp)
  xw1p = _tc_xw1(x_r, w1bd)
  y1p, dinv = _tc_y1(degp, xw1p)
  (acc1p,) = _seg_sum([y1p], srcp, dstp)
  y2l, y2r, xw2l, xw2r = _tc_layer2(acc1p, xw1p, dinv, b1t, w2lbd, w2rbd)
  (acc2lp,) = _seg_sum([y2l], srcp, dstp)
  (acc2rp,) = _seg_sum([y2r], srcp, dstp)
  out8 = _tc_out(acc2lp, acc2rp, xw2l, xw2r, dinv,
                 b2lt, b2rt, w3lt, w3rt, sel, b3r)
  return out8.reshape(-1)[:N].reshape(N, 1)


def kernel(x, edge_index, W1, b1, W2, b2, W3, b3):
  return _run(x, edge_index, W1, b1, W2, b2, W3, b3)


# speedup vs baseline: 53.9768x; 1.0567x over previous
import functools

import jax
import jax.numpy as jnp
from jax import lax
from jax.experimental import pallas as pl
from jax.experimental.pallas import tpu as pltpu
from jax.experimental.pallas import tpu_sc as plsc

N = 10000
E = 320000
D = 128
H1 = 16
H2 = 32

NC = 2
NS = 16
CH = 128
NCH = 80
EPT = CH * NCH
E_PAD = EPT * NC * NS
N_PAD = 10240
NP8 = N_PAD // 8
TPW = NP8 // NS
RPW = N_PAD // NS

_mesh = plsc.VectorSubcoreMesh(core_axis_name="c", subcore_axis_name="s")
_sc_params = pltpu.CompilerParams(use_tc_tiling_on_sc=False)


def _unpack_rows(v128, v16):
  @pl.loop(0, TPW)
  def _(g):
    for k in range(8):
      v16[g * 8 + k, :] = v128[g, pl.ds(k * 16, 16)]


def _pack_rows(v16, v128):
  @pl.loop(0, TPW)
  def _(g):
    for k in range(8):
      v128[g, pl.ds(k * 16, 16)] = v16[g * 8 + k, :]


def _zero_v16(v16):
  z = jnp.zeros((16,), jnp.float32)
  @pl.loop(0, RPW)
  def _(i):
    v16[i, :] = z


DEPTH = 4


def _seg_sum(tables, srcp, dstp, dummy):
  nt = len(tables)

  @functools.partial(
      pl.kernel,
      out_type=[jax.ShapeDtypeStruct((NC, NP8, 128), jnp.float32)] * nt,
      mesh=_mesh,
      compiler_params=_sc_params,
      scratch_types=(
          [pltpu.VMEM((NCH, CH), jnp.int32)] * 2
          + [pltpu.VMEM((TPW, 128), jnp.float32)]
          + [pltpu.VMEM((RPW, 16), jnp.float32)]
          + [pltpu.VMEM((CH, 16), jnp.float32)] * (DEPTH * nt)
          + [pltpu.VMEM_SHARED((N_PAD, 16), jnp.float32)] * (2 * nt)
          + [pltpu.SemaphoreType.DMA] * (2 * DEPTH * nt + 2)
      ),
  )
  def k(*refs):
    y_hbm = refs[:nt]
    src_hbm, dst_hbm, dummy_hbm = refs[nt], refs[nt + 1], refs[nt + 2]
    out_hbm = refs[nt + 3:2 * nt + 3]
    r = 2 * nt + 3
    src_v, dst_v = refs[r], refs[r + 1]
    v128, v16 = refs[r + 2], refs[r + 3]
    r += 4
    rows = [[refs[r + p * nt + t] for t in range(nt)] for p in range(DEPTH)]
    r += DEPTH * nt
    table_sh = refs[r:r + nt]
    acc_sh = refs[r + nt:r + 2 * nt]
    r += 2 * nt
    sem_g = [[refs[r + p * nt + t] for t in range(nt)] for p in range(DEPTH)]
    r += DEPTH * nt
    sem_s = [[refs[r + p * nt + t] for t in range(nt)] for p in range(DEPTH)]
    r += DEPTH * nt
    sem_i, sem_c = refs[r], refs[r + 1]

    c = lax.axis_index("c")
    s = lax.axis_index("s")
    prow0 = s * TPW
    nrow0 = s * RPW

    h1 = pltpu.async_copy(src_hbm.at[c, s], src_v, sem_i)
    h2 = pltpu.async_copy(dst_hbm.at[c, s], dst_v, sem_i)

    for t in range(nt):
      pltpu.async_copy(
          y_hbm[t].at[pl.ds(prow0, TPW)], v128, sem_c).wait()
      _unpack_rows(v128, v16)
      pltpu.async_copy(
          v16, table_sh[t].at[pl.ds(nrow0, RPW)], sem_c).wait()

    _zero_v16(v16)
    for t in range(nt):
      pltpu.async_copy(
          v16, acc_sh[t].at[pl.ds(nrow0, RPW)], sem_c).wait()

    h1.wait()
    h2.wait()
    plsc.subcore_barrier()

    @pl.loop(0, NCH, step=DEPTH)
    def _(j):
      hg = [[pltpu.async_copy(table_sh[t].at[src_v.at[j + p]], rows[p][t],
                              sem_g[p][t]) for t in range(nt)]
            for p in range(DEPTH)]
      for p in range(DEPTH):
        for t in range(nt):
          hg[p][t].wait()
          pltpu.sync_copy(rows[p][t], acc_sh[t].at[dst_v.at[j + p]],
                          add=True)

    plsc.subcore_barrier()

    for t in range(nt):
      pltpu.async_copy(
          acc_sh[t].at[pl.ds(nrow0, RPW)], v16, sem_c).wait()
      _pack_rows(v16, v128)
      pltpu.async_copy(
          v128, out_hbm[t].at[c, pl.ds(prow0, TPW)], sem_c).wait()

  outs = k(*tables, srcp, dstp, dummy)
  return list(outs) if isinstance(outs, (list, tuple)) else [outs]


def _deg_count(dstp):
  @functools.partial(
      pl.kernel,
      out_type=jax.ShapeDtypeStruct((NC, NP8, 128), jnp.float32),
      mesh=_mesh,
      compiler_params=_sc_params,
      scratch_types=[
          pltpu.VMEM((NCH, CH), jnp.int32),
          pltpu.VMEM((TPW, 128), jnp.float32),
          pltpu.VMEM((RPW, 16), jnp.float32),
          pltpu.VMEM((CH, 16), jnp.float32),
          pltpu.VMEM_SHARED((N_PAD, 16), jnp.float32),
          pltpu.SemaphoreType.DMA,
      ],
  )
  def k(dst_hbm, out_hbm, dst_v, v128, v16, ones_v, acc_sh, sem):
    c = lax.axis_index("c")
    s = lax.axis_index("s")
    prow0 = s * TPW
    nrow0 = s * RPW

    h1 = pltpu.async_copy(dst_hbm.at[c, s], dst_v, sem)

    one = jnp.ones((16,), jnp.float32)
    @pl.loop(0, CH)
    def _(i):
      ones_v[i, :] = one

    _zero_v16(v16)
    pltpu.async_copy(v16, acc_sh.at[pl.ds(nrow0, RPW)], sem).wait()
    h1.wait()
    plsc.subcore_barrier()

    @pl.loop(0, NCH)
    def _(j):
      pltpu.sync_copy(ones_v, acc_sh.at[dst_v.at[j]], add=True)

    plsc.subcore_barrier()

    pltpu.async_copy(acc_sh.at[pl.ds(nrow0, RPW)], v16, sem).wait()
    _pack_rows(v16, v128)
    pltpu.async_copy(v128, out_hbm.at[c, pl.ds(prow0, TPW)], sem).wait()

  return k(dstp)


_HP = lax.Precision.HIGHEST
_BF = jnp.bfloat16


def _bdot(a, b):
  return jnp.dot(a.astype(_BF), b.astype(_BF),
                 preferred_element_type=jnp.float32)


def _tc_xw1(x_r, w1bd):
  def body(x_ref, w_ref, o_ref):
    o_ref[...] = _bdot(x_ref[...], w_ref[...])

  return pl.pallas_call(
      body,
      out_shape=jax.ShapeDtypeStruct((NP8, 128), jnp.float32),
  )(x_r, w1bd)


def _tc_y1(degp, xw1p):
  def body(d_ref, xw_ref, y_ref, dinv_ref):
    dinv = lax.rsqrt(d_ref[0] + d_ref[1] + 1.0)
    dinv_ref[...] = dinv
    y_ref[...] = dinv * xw_ref[...]

  return pl.pallas_call(
      body,
      out_shape=[jax.ShapeDtypeStruct((NP8, 128), jnp.float32)] * 2,
  )(degp, xw1p)


def _tc_layer2(acc1p, xw1p, dinv, b1t, w2lbd, w2rbd):
  def body(a_ref, xw_ref, d_ref, b1_ref, wl_ref, wr_ref,
           y2l_ref, y2r_ref, xw2l_ref, xw2r_ref):
    dv = d_ref[...]
    h1 = dv * (a_ref[0] + a_ref[1]) + dv * dv * xw_ref[...] + b1_ref[...]
    h1 = jnp.maximum(h1, 0.0)
    xw2l = _bdot(h1, wl_ref[...])
    xw2r = _bdot(h1, wr_ref[...])
    xw2l_ref[...] = xw2l
    xw2r_ref[...] = xw2r
    y2l_ref[...] = dv * xw2l
    y2r_ref[...] = dv * xw2r

  return pl.pallas_call(
      body,
      out_shape=[jax.ShapeDtypeStruct((NP8, 128), jnp.float32)] * 4,
  )(acc1p, xw1p, dinv, b1t, w2lbd, w2rbd)


def _tc_out(acc2lp, acc2rp, xw2l, xw2r, dinv, b2lt, b2rt, w3lt, w3rt, sel, b3):
  def body(al_ref, ar_ref, xl_ref, xr_ref, d_ref, b2l_ref, b2r_ref,
           w3l_ref, w3r_ref, s_ref, b3_ref, o_ref):
    dv = d_ref[...]
    dv2 = dv * dv
    h2l = dv * (al_ref[0] + al_ref[1]) + dv2 * xl_ref[...] + b2l_ref[...]
    h2r = dv * (ar_ref[0] + ar_ref[1]) + dv2 * xr_ref[...] + b2r_ref[...]
    w3l16 = w3l_ref[...].astype(_BF).astype(jnp.float32)
    w3r16 = w3r_ref[...].astype(_BF).astype(jnp.float32)
    t = (h2l.astype(_BF).astype(jnp.float32) * w3l16
         + h2r.astype(_BF).astype(jnp.float32) * w3r16)
    o_ref[...] = jnp.dot(t, s_ref[...], preferred_element_type=jnp.float32,
                         precision=_HP) + b3_ref[0, 0]

  return pl.pallas_call(
      body,
      out_shape=jax.ShapeDtypeStruct((NP8, 8), jnp.float32),
  )(acc2lp, acc2rp, xw2l, xw2r, dinv, b2lt, b2rt, w3lt, w3rt, sel, b3)


def _blockdiag8(W):
  din, dout = W.shape
  z = jnp.zeros((8 * din, 8 * dout), W.dtype)
  for k in range(8):
    z = lax.dynamic_update_slice(z, W, (din * k, dout * k))
  return z


@jax.jit
def _run(x, edge_index, W1, b1, W2, b2, W3, b3):
  src = edge_index[0]
  dst = edge_index[1]
  pad = E_PAD - E
  srcp = jnp.concatenate(
      [src, jnp.zeros((pad,), src.dtype)]).reshape(NC, NS, NCH, CH)
  dstp = jnp.concatenate(
      [dst, jnp.full((pad,), N, dst.dtype)]).reshape(NC, NS, NCH, CH)

  x_r = jnp.pad(x.reshape(N // 8, 8 * D), ((0, NP8 - N // 8), (0, 0)))
  w1bd = _blockdiag8(W1)
  w2lbd = _blockdiag8(W2[:, :16])
  w2rbd = _blockdiag8(W2[:, 16:])
  b1t = jnp.tile(b1, 8).reshape(1, 128)
  b2lt = jnp.tile(b2[:16], 8).reshape(1, 128)
  b2rt = jnp.tile(b2[16:], 8).reshape(1, 128)
  w3lt = jnp.tile(W3[:16, 0], 8).reshape(1, 128)
  w3rt = jnp.tile(W3[16:, 0], 8).reshape(1, 128)
  sel = jnp.repeat(jnp.eye(8, dtype=jnp.float32), 16, axis=0)
  b3r = b3.reshape(1, 1)

  dummy = jnp.zeros((CH, 16), jnp.float32)
  degp = _deg_count(dstp)
  xw1p = _tc_xw1(x_r, w1bd)
  y1p, dinv = _tc_y1(degp, xw1p)
  (acc1p,) = _seg_sum([y1p], srcp, dstp, dummy)
  y2l, y2r, xw2l, xw2r = _tc_layer2(acc1p, xw1p, dinv, b1t, w2lbd, w2rbd)
  (acc2lp,) = _seg_sum([y2l], srcp, dstp, dummy)
  (acc2rp,) = _seg_sum([y2r], srcp, dstp, dummy)
  out8 = _tc_out(acc2lp, acc2rp, xw2l, xw2r, dinv,
                 b2lt, b2rt, w3lt, w3rt, sel, b3r)
  return out8.reshape(-1)[:N].reshape(N, 1)


def kernel(x, edge_index, W1, b1, W2, b2, W3, b3):
  return _run(x, edge_index, W1, b1, W2, b2, W3, b3)


# speedup vs baseline: 55.7650x; 1.0331x over previous
import functools

import jax
import jax.numpy as jnp
from jax import lax
from jax.experimental import pallas as pl
from jax.experimental.pallas import tpu as pltpu
from jax.experimental.pallas import tpu_sc as plsc

N = 10000
E = 320000
D = 128
H1 = 16
H2 = 32

NC = 2
NS = 16
CH = 128
NCH = 80
EPT = CH * NCH
E_PAD = EPT * NC * NS
N_PAD = 10240
NP8 = N_PAD // 8
TPW = NP8 // NS
RPW = N_PAD // NS

_mesh = plsc.VectorSubcoreMesh(core_axis_name="c", subcore_axis_name="s")
_sc_params = pltpu.CompilerParams(use_tc_tiling_on_sc=False)


def _unpack_rows(v128, v16):
  @pl.loop(0, TPW)
  def _(g):
    for k in range(8):
      v16[g * 8 + k, :] = v128[g, pl.ds(k * 16, 16)]


def _pack_rows(v16, v128):
  @pl.loop(0, TPW)
  def _(g):
    for k in range(8):
      v128[g, pl.ds(k * 16, 16)] = v16[g * 8 + k, :]


def _zero_v16(v16):
  z = jnp.zeros((16,), jnp.float32)
  @pl.loop(0, RPW)
  def _(i):
    v16[i, :] = z


DEPTH = 4


def _seg_sum(tables, srcp, dstp, dummy):
  nt = len(tables)

  @functools.partial(
      pl.kernel,
      out_type=[jax.ShapeDtypeStruct((NC, NP8, 128), jnp.float32)] * nt,
      mesh=_mesh,
      compiler_params=_sc_params,
      scratch_types=(
          [pltpu.VMEM((NCH, CH), jnp.int32)] * 2
          + [pltpu.VMEM((TPW, 128), jnp.float32)]
          + [pltpu.VMEM((RPW, 16), jnp.float32)]
          + [pltpu.VMEM((CH, 16), jnp.float32)] * (DEPTH * nt)
          + [pltpu.VMEM_SHARED((N_PAD, 16), jnp.float32)] * (2 * nt)
          + [pltpu.SemaphoreType.DMA] * (2 * DEPTH * nt + 2)
      ),
  )
  def k(*refs):
    y_hbm = refs[:nt]
    src_hbm, dst_hbm, dummy_hbm = refs[nt], refs[nt + 1], refs[nt + 2]
    out_hbm = refs[nt + 3:2 * nt + 3]
    r = 2 * nt + 3
    src_v, dst_v = refs[r], refs[r + 1]
    v128, v16 = refs[r + 2], refs[r + 3]
    r += 4
    rows = [[refs[r + p * nt + t] for t in range(nt)] for p in range(DEPTH)]
    r += DEPTH * nt
    table_sh = refs[r:r + nt]
    acc_sh = refs[r + nt:r + 2 * nt]
    r += 2 * nt
    sem_g = [[refs[r + p * nt + t] for t in range(nt)] for p in range(DEPTH)]
    r += DEPTH * nt
    sem_s = [[refs[r + p * nt + t] for t in range(nt)] for p in range(DEPTH)]
    r += DEPTH * nt
    sem_i, sem_c = refs[r], refs[r + 1]

    c = lax.axis_index("c")
    s = lax.axis_index("s")
    prow0 = s * TPW
    nrow0 = s * RPW

    h1 = pltpu.async_copy(src_hbm.at[c, s], src_v, sem_i)
    h2 = pltpu.async_copy(dst_hbm.at[c, s], dst_v, sem_i)

    for t in range(nt):
      pltpu.async_copy(
          y_hbm[t].at[pl.ds(prow0, TPW)], v128, sem_c).wait()
      _unpack_rows(v128, v16)
      pltpu.async_copy(
          v16, table_sh[t].at[pl.ds(nrow0, RPW)], sem_c).wait()

    _zero_v16(v16)
    for t in range(nt):
      pltpu.async_copy(
          v16, acc_sh[t].at[pl.ds(nrow0, RPW)], sem_c).wait()

    h1.wait()
    h2.wait()
    plsc.subcore_barrier()

    @pl.loop(0, NCH, step=DEPTH)
    def _(j):
      hg = [[pltpu.async_copy(table_sh[t].at[src_v.at[j + p]], rows[p][t],
                              sem_g[p][t]) for t in range(nt)]
            for p in range(DEPTH)]
      hs = [[None] * nt for _ in range(DEPTH)]
      for p in range(DEPTH):
        for t in range(nt):
          hg[p][t].wait()
          hs[p][t] = pltpu.async_copy(
              rows[p][t], acc_sh[t].at[dst_v.at[j + p]], sem_s[p][t],
              add=True)
      for p in range(DEPTH):
        for t in range(nt):
          hs[p][t].wait()

    plsc.subcore_barrier()

    for t in range(nt):
      pltpu.async_copy(
          acc_sh[t].at[pl.ds(nrow0, RPW)], v16, sem_c).wait()
      _pack_rows(v16, v128)
      pltpu.async_copy(
          v128, out_hbm[t].at[c, pl.ds(prow0, TPW)], sem_c).wait()

  outs = k(*tables, srcp, dstp, dummy)
  return list(outs) if isinstance(outs, (list, tuple)) else [outs]


def _deg_count(dstp):
  @functools.partial(
      pl.kernel,
      out_type=jax.ShapeDtypeStruct((NC, NP8, 128), jnp.float32),
      mesh=_mesh,
      compiler_params=_sc_params,
      scratch_types=[
          pltpu.VMEM((NCH, CH), jnp.int32),
          pltpu.VMEM((TPW, 128), jnp.float32),
          pltpu.VMEM((RPW, 16), jnp.float32),
          pltpu.VMEM((CH, 16), jnp.float32),
          pltpu.VMEM_SHARED((N_PAD, 16), jnp.float32),
          pltpu.SemaphoreType.DMA,
      ],
  )
  def k(dst_hbm, out_hbm, dst_v, v128, v16, ones_v, acc_sh, sem):
    c = lax.axis_index("c")
    s = lax.axis_index("s")
    prow0 = s * TPW
    nrow0 = s * RPW

    h1 = pltpu.async_copy(dst_hbm.at[c, s], dst_v, sem)

    one = jnp.ones((16,), jnp.float32)
    @pl.loop(0, CH)
    def _(i):
      ones_v[i, :] = one

    _zero_v16(v16)
    pltpu.async_copy(v16, acc_sh.at[pl.ds(nrow0, RPW)], sem).wait()
    h1.wait()
    plsc.subcore_barrier()

    @pl.loop(0, NCH, step=16)
    def _(j):
      hs = [pltpu.async_copy(ones_v, acc_sh.at[dst_v.at[j + p]], sem,
                             add=True) for p in range(16)]
      for h in hs:
        h.wait()

    plsc.subcore_barrier()

    pltpu.async_copy(acc_sh.at[pl.ds(nrow0, RPW)], v16, sem).wait()
    _pack_rows(v16, v128)
    pltpu.async_copy(v128, out_hbm.at[c, pl.ds(prow0, TPW)], sem).wait()

  return k(dstp)


_HP = lax.Precision.HIGHEST
_BF = jnp.bfloat16


def _bdot(a, b):
  return jnp.dot(a.astype(_BF), b.astype(_BF),
                 preferred_element_type=jnp.float32)


def _tc_xw1(x_r, w1bd):
  def body(x_ref, w_ref, o_ref):
    o_ref[...] = _bdot(x_ref[...], w_ref[...])

  return pl.pallas_call(
      body,
      out_shape=jax.ShapeDtypeStruct((NP8, 128), jnp.float32),
  )(x_r, w1bd)


def _tc_y1(degp, xw1p):
  def body(d_ref, xw_ref, y_ref, dinv_ref):
    dinv = lax.rsqrt(d_ref[0] + d_ref[1] + 1.0)
    dinv_ref[...] = dinv
    y_ref[...] = dinv * xw_ref[...]

  return pl.pallas_call(
      body,
      out_shape=[jax.ShapeDtypeStruct((NP8, 128), jnp.float32)] * 2,
  )(degp, xw1p)


def _tc_layer2(acc1p, xw1p, dinv, b1t, w2lbd, w2rbd):
  def body(a_ref, xw_ref, d_ref, b1_ref, wl_ref, wr_ref,
           y2l_ref, y2r_ref, xw2l_ref, xw2r_ref):
    dv = d_ref[...]
    h1 = dv * (a_ref[0] + a_ref[1]) + dv * dv * xw_ref[...] + b1_ref[...]
    h1 = jnp.maximum(h1, 0.0)
    xw2l = _bdot(h1, wl_ref[...])
    xw2r = _bdot(h1, wr_ref[...])
    xw2l_ref[...] = xw2l
    xw2r_ref[...] = xw2r
    y2l_ref[...] = dv * xw2l
    y2r_ref[...] = dv * xw2r

  return pl.pallas_call(
      body,
      out_shape=[jax.ShapeDtypeStruct((NP8, 128), jnp.float32)] * 4,
  )(acc1p, xw1p, dinv, b1t, w2lbd, w2rbd)


def _tc_out(acc2lp, acc2rp, xw2l, xw2r, dinv, b2lt, b2rt, w3lt, w3rt, sel, b3):
  def body(al_ref, ar_ref, xl_ref, xr_ref, d_ref, b2l_ref, b2r_ref,
           w3l_ref, w3r_ref, s_ref, b3_ref, o_ref):
    dv = d_ref[...]
    dv2 = dv * dv
    h2l = dv * (al_ref[0] + al_ref[1]) + dv2 * xl_ref[...] + b2l_ref[...]
    h2r = dv * (ar_ref[0] + ar_ref[1]) + dv2 * xr_ref[...] + b2r_ref[...]
    w3l16 = w3l_ref[...].astype(_BF).astype(jnp.float32)
    w3r16 = w3r_ref[...].astype(_BF).astype(jnp.float32)
    t = (h2l.astype(_BF).astype(jnp.float32) * w3l16
         + h2r.astype(_BF).astype(jnp.float32) * w3r16)
    o_ref[...] = jnp.dot(t, s_ref[...], preferred_element_type=jnp.float32,
                         precision=_HP) + b3_ref[0, 0]

  return pl.pallas_call(
      body,
      out_shape=jax.ShapeDtypeStruct((NP8, 8), jnp.float32),
  )(acc2lp, acc2rp, xw2l, xw2r, dinv, b2lt, b2rt, w3lt, w3rt, sel, b3)


def _blockdiag8(W):
  din, dout = W.shape
  z = jnp.zeros((8 * din, 8 * dout), W.dtype)
  for k in range(8):
    z = lax.dynamic_update_slice(z, W, (din * k, dout * k))
  return z


@jax.jit
def _run(x, edge_index, W1, b1, W2, b2, W3, b3):
  src = edge_index[0]
  dst = edge_index[1]
  pad = E_PAD - E
  srcp = jnp.concatenate(
      [src, jnp.zeros((pad,), src.dtype)]).reshape(NC, NS, NCH, CH)
  dstp = jnp.concatenate(
      [dst, jnp.full((pad,), N, dst.dtype)]).reshape(NC, NS, NCH, CH)

  x_r = jnp.pad(x.reshape(N // 8, 8 * D), ((0, NP8 - N // 8), (0, 0)))
  w1bd = _blockdiag8(W1)
  w2lbd = _blockdiag8(W2[:, :16])
  w2rbd = _blockdiag8(W2[:, 16:])
  b1t = jnp.tile(b1, 8).reshape(1, 128)
  b2lt = jnp.tile(b2[:16], 8).reshape(1, 128)
  b2rt = jnp.tile(b2[16:], 8).reshape(1, 128)
  w3lt = jnp.tile(W3[:16, 0], 8).reshape(1, 128)
  w3rt = jnp.tile(W3[16:, 0], 8).reshape(1, 128)
  sel = jnp.repeat(jnp.eye(8, dtype=jnp.float32), 16, axis=0)
  b3r = b3.reshape(1, 1)

  dummy = jnp.zeros((CH, 16), jnp.float32)
  degp = _deg_count(dstp)
  xw1p = _tc_xw1(x_r, w1bd)
  y1p, dinv = _tc_y1(degp, xw1p)
  (acc1p,) = _seg_sum([y1p], srcp, dstp, dummy)
  y2l, y2r, xw2l, xw2r = _tc_layer2(acc1p, xw1p, dinv, b1t, w2lbd, w2rbd)
  acc2lp, acc2rp = _seg_sum([y2l, y2r], srcp, dstp, dummy)
  out8 = _tc_out(acc2lp, acc2rp, xw2l, xw2r, dinv,
                 b2lt, b2rt, w3lt, w3rt, sel, b3r)
  return out8.reshape(-1)[:N].reshape(N, 1)


def kernel(x, edge_index, W1, b1, W2, b2, W3, b3):
  return _run(x, edge_index, W1, b1, W2, b2, W3, b3)
